# baseline (speedup 1.0000x reference)
import functools

import jax
import jax.numpy as jnp
from jax import lax
from jax.experimental import pallas as pl
from jax.experimental.pallas import tpu as pltpu
from jax.experimental.pallas import tpu_sc as plsc

NUM_CORES = 2
NUM_SUBCORES = 16
LANES = 16
QB = 256
NB = 2

B = 4096
L = 200
D = 32
VB = D // 8


@jax.jit
def _sc_embed(xt, t4, posb):
    nw = NUM_CORES * NUM_SUBCORES
    nq = B // QB
    units = L * nq
    upw = units // nw
    d2 = (B // 128) * 8 * 128
    bbl_per_q = QB // 128
    osz = bbl_per_q * 8 * 128

    mesh = plsc.VectorSubcoreMesh(
        core_axis_name="c", subcore_axis_name="s",
        num_cores=NUM_CORES, num_subcores=NUM_SUBCORES,
    )

    @functools.partial(
        pl.kernel,
        mesh=mesh,
        out_type=jax.ShapeDtypeStruct((L, VB, d2), jnp.float32),
        scratch_types=[
            [pltpu.VMEM((QB,), jnp.int32) for _ in range(NB)],
            [pltpu.VMEM((QB,), jnp.int32) for _ in range(NB)],
            [pltpu.VMEM((QB, 128), jnp.float32) for _ in range(NB)],
            [pltpu.VMEM((VB, osz), jnp.float32) for _ in range(NB)],
            [pltpu.VMEM((D * LANES,), jnp.float32) for _ in range(NB)],
            [pltpu.SemaphoreType.DMA for _ in range(NB)],
            [[pltpu.SemaphoreType.DMA for _ in range(VB)] for _ in range(NB)],
        ],
        compiler_params=pltpu.CompilerParams(
            use_tc_tiling_on_sc=False, needs_layout_passes=False),
    )
    def k(xt_h, t4_h, posb_h, out_h, idxb, idx4b, rows, obuf, posu, gsem, osem):
        wid = lax.axis_index("s") * NUM_CORES + lax.axis_index("c")
        u0 = wid * upw
        iota = lax.iota(jnp.int32, LANES)

        def stage(j, b):
            l, q = (u0 + j) // nq, (u0 + j) % nq
            pltpu.sync_copy(xt_h.at[l, pl.ds(q * QB, QB)], idxb[b])
            pltpu.sync_copy(posb_h.at[l], posu[b])

            def sh(m, c):
                sl = pl.ds(m * LANES, LANES)
                idx4b[b][sl] = lax.shift_right_logical(idxb[b][sl], 2)
                return c

            lax.fori_loop(0, QB // LANES, sh, 0)
            pltpu.async_copy(t4_h.at[idx4b[b]], rows[b], gsem[b])

        def wait_gather(b):
            pltpu.make_async_copy(t4_h.at[idx4b[b]], rows[b], gsem[b]).wait()

        def transpose_add(j, b):

            def mbody(m, c):
                sl = pl.ds(m * LANES, LANES)
                iv = idxb[b][sl]
                colb = lax.shift_left(lax.bitwise_and(iv, 3), 5)
                row = iota + m * LANES
                base = lax.shift_right_logical(m, 3) * 1024 + \
                    lax.bitwise_and(m, 7) * LANES

                def vbody(v, c2):
                    p = posu[b][pl.ds(v * LANES, LANES)]
                    vals = plsc.load_gather(rows[b], [row, colb + v])
                    vblk = lax.shift_right_logical(v, 3)
                    off = base + lax.bitwise_and(v, 7) * 128
                    obuf[b][vblk, pl.ds(off, LANES)] = vals + p
                    return c2

                lax.fori_loop(0, D, vbody, 0, unroll=4)
                return c

            lax.fori_loop(0, QB // LANES, mbody, 0)

        def out_copies(j, b):
            l, q = (u0 + j) // nq, (u0 + j) % nq
            return [
                pltpu.make_async_copy(
                    obuf[b].at[vb], out_h.at[l, vb, pl.ds(q * osz, osz)],
                    osem[b][vb])
                for vb in range(VB)
            ]

        def start_out(j, b):
            for c in out_copies(j, b):
                c.start()

        def wait_out(j, b):
            for c in out_copies(j, b):
                c.wait()

        stage(0, 0)
        stage(1, 1)
        for j in range(NB):
            b = j % NB
            wait_gather(b)
            transpose_add(j, b)
            stage(j + NB, b)
            start_out(j, b)

        def group(g, c):
            for b in range(NB):
                j = g * NB + b
                wait_gather(b)
                wait_out(j - NB, b)
                transpose_add(j, b)
                stage(j + NB, b)
                start_out(j, b)
            return c

        lax.fori_loop(1, upw // NB - 1, group, 0)

        for j in range(upw - NB, upw):
            b = j % NB
            wait_gather(b)
            wait_out(j - NB, b)
            transpose_add(j, b)
            start_out(j, b)
        for j in range(upw - NB, upw):
            wait_out(j, j % NB)

    return k(xt, t4, posb)


def kernel(x, table, pos_emb):
    xt = jnp.transpose(x).astype(jnp.int32)
    t4 = jnp.reshape(table, (table.shape[0] // 4, 4 * table.shape[1]))
    posb = jnp.broadcast_to(pos_emb[:L, :, None], (L, D, LANES)).reshape(L, D * LANES)
    o5 = _sc_embed(xt, t4, posb)
    o = o5.reshape(L, VB, B // 128, 8, 128).transpose(2, 4, 0, 1, 3)
    return o.reshape(B, L, D)


# speedup vs baseline: 1.2643x; 1.2643x over previous
import functools

import jax
import jax.numpy as jnp
from jax import lax
from jax.experimental import pallas as pl
from jax.experimental.pallas import tpu as pltpu
from jax.experimental.pallas import tpu_sc as plsc

NUM_CORES = 2
NUM_SUBCORES = 16
LANES = 16
QB = 512
NB = 2

B = 4096
L = 200
D = 32
VB = D // 8


@jax.jit
def _sc_embed(xt, table, pos):
    nw = NUM_CORES * NUM_SUBCORES
    nq = B // QB
    units = L * nq
    upw = units // nw
    d2 = (B // 128) * 8 * 128
    osz = (QB // 128) * 8 * 128

    mesh = plsc.VectorSubcoreMesh(
        core_axis_name="c", subcore_axis_name="s",
        num_cores=NUM_CORES, num_subcores=NUM_SUBCORES,
    )

    @functools.partial(
        pl.kernel,
        mesh=mesh,
        out_type=jax.ShapeDtypeStruct((L, VB, d2), jnp.float32),
        scratch_types=[
            [pltpu.VMEM((QB,), jnp.int32) for _ in range(NB)],
            [pltpu.VMEM((QB, D), jnp.float32) for _ in range(NB)],
            [pltpu.VMEM((VB * osz,), jnp.float32) for _ in range(NB)],
            pltpu.VMEM((L, D), jnp.float32),
            [pltpu.SemaphoreType.DMA for _ in range(NB)],
            [[pltpu.SemaphoreType.DMA for _ in range(VB)] for _ in range(NB)],
        ],
        compiler_params=pltpu.CompilerParams(
            use_tc_tiling_on_sc=False, needs_layout_passes=False),
    )
    def k(xt_h, t_h, pos_h, out_h, idxb, rows, obuf, pos_v, gsem, osem):
        wid = lax.axis_index("s") * NUM_CORES + lax.axis_index("c")
        u0 = wid * upw
        iota = lax.iota(jnp.int32, LANES)
        pat0 = lax.shift_right_logical(iota, 3) * osz + \
            lax.bitwise_and(iota, 7) * 128
        pat1 = pat0 + 2 * osz
        pltpu.sync_copy(pos_h, pos_v)

        def stage(j, b):
            l, q = (u0 + j) // nq, (u0 + j) % nq
            pltpu.sync_copy(xt_h.at[l, pl.ds(q * QB, QB)], idxb[b])
            pltpu.async_copy(t_h.at[idxb[b]], rows[b], gsem[b])

        def wait_gather(b):
            pltpu.make_async_copy(t_h.at[idxb[b]], rows[b], gsem[b]).wait()

        def transpose_add(j, b):
            l = (u0 + j) // nq
            p0 = pos_v[l, pl.ds(0, LANES)]
            p1 = pos_v[l, pl.ds(LANES, LANES)]

            def jbody(jj, c):
                off = lax.shift_right_logical(jj, 7) * 1024 + \
                    lax.bitwise_and(jj, 127)
                v0 = rows[b][jj, pl.ds(0, LANES)]
                v1 = rows[b][jj, pl.ds(LANES, LANES)]
                plsc.store_scatter(obuf[b], [pat0 + off], v0 + p0)
                plsc.store_scatter(obuf[b], [pat1 + off], v1 + p1)
                return c

            lax.fori_loop(0, QB, jbody, 0, unroll=4)

        def out_copies(j, b):
            l, q = (u0 + j) // nq, (u0 + j) % nq
            return [
                pltpu.make_async_copy(
                    obuf[b].at[pl.ds(vb * osz, osz)],
                    out_h.at[l, vb, pl.ds(q * osz, osz)],
                    osem[b][vb])
                for vb in range(VB)
            ]

        def start_out(j, b):
            for c in out_copies(j, b):
                c.start()

        def wait_out(j, b):
            for c in out_copies(j, b):
                c.wait()

        stage(0, 0)
        stage(1, 1)
        for j in range(NB):
            b = j % NB
            wait_gather(b)
            transpose_add(j, b)
            stage(j + NB, b)
            start_out(j, b)

        def group(g, c):
            for b in range(NB):
                j = g * NB + b
                wait_gather(b)
                wait_out(j - NB, b)
                transpose_add(j, b)
                stage(j + NB, b)
                start_out(j, b)
            return c

        lax.fori_loop(1, upw // NB - 1, group, 0)

        for j in range(upw - NB, upw):
            b = j % NB
            wait_gather(b)
            wait_out(j - NB, b)
            transpose_add(j, b)
            start_out(j, b)
        for j in range(upw - NB, upw):
            wait_out(j, j % NB)

    return k(xt, table, pos)


def kernel(x, table, pos_emb):
    xt = jnp.transpose(x).astype(jnp.int32)
    o5 = _sc_embed(xt, table, pos_emb[:L])
    o = o5.reshape(L, VB, B // 128, 8, 128).transpose(2, 4, 0, 1, 3)
    return o.reshape(B, L, D)


# speedup vs baseline: 1.4005x; 1.1077x over previous
import functools

import jax
import jax.numpy as jnp
from jax import lax
from jax.experimental import pallas as pl
from jax.experimental.pallas import tpu as pltpu
from jax.experimental.pallas import tpu_sc as plsc

NUM_CORES = 2
NUM_SUBCORES = 16
LANES = 16
QB = 512
NB = 2

B = 4096
L = 200
D = 32
VB = D // 8


@jax.jit
def _sc_embed(xt, table, pos):
    nw = NUM_CORES * NUM_SUBCORES
    nq = B // QB
    units = L * nq
    upw = units // nw
    d2 = (B // 128) * 8 * 128
    osz = (QB // 128) * 8 * 128

    mesh = plsc.VectorSubcoreMesh(
        core_axis_name="c", subcore_axis_name="s",
        num_cores=NUM_CORES, num_subcores=NUM_SUBCORES,
    )

    @functools.partial(
        pl.kernel,
        mesh=mesh,
        out_type=jax.ShapeDtypeStruct((L, VB, d2), jnp.float32),
        scratch_types=[
            [pltpu.VMEM((QB,), jnp.int32) for _ in range(NB)],
            [pltpu.VMEM((QB, D), jnp.float32) for _ in range(NB)],
            [pltpu.VMEM((VB * osz,), jnp.float32) for _ in range(NB)],
            pltpu.VMEM((L, D), jnp.float32),
            [pltpu.SemaphoreType.DMA for _ in range(NB)],
            [[pltpu.SemaphoreType.DMA for _ in range(VB)] for _ in range(NB)],
        ],
        compiler_params=pltpu.CompilerParams(
            use_tc_tiling_on_sc=False, needs_layout_passes=False),
    )
    def k(xt_h, t_h, pos_h, out_h, idxb, rows, obuf, pos_v, gsem, osem):
        wid = lax.axis_index("s") * NUM_CORES + lax.axis_index("c")
        u0 = wid * upw
        iota = lax.iota(jnp.int32, LANES)
        pat0 = lax.shift_right_logical(iota, 3) * osz + \
            lax.bitwise_and(iota, 7) * 128
        pat1 = pat0 + 2 * osz
        pltpu.sync_copy(pos_h, pos_v)

        def stage(j, b):
            l, q = (u0 + j) // nq, (u0 + j) % nq
            pltpu.sync_copy(xt_h.at[l, pl.ds(q * QB, QB)], idxb[b])
            pltpu.async_copy(t_h.at[idxb[b]], rows[b], gsem[b])

        def wait_gather(b):
            pltpu.make_async_copy(t_h.at[idxb[b]], rows[b], gsem[b]).wait()

        def transpose_add(j, b):
            l = (u0 + j) // nq
            p0 = pos_v[l, pl.ds(0, LANES)]
            p1 = pos_v[l, pl.ds(LANES, LANES)]

            @plsc.parallel_loop(0, QB, unroll=8)
            def jbody(jj):
                off = lax.shift_right_logical(jj, 7) * 1024 + \
                    lax.bitwise_and(jj, 127)
                v0 = rows[b][jj, pl.ds(0, LANES)]
                v1 = rows[b][jj, pl.ds(LANES, LANES)]
                plsc.store_scatter(obuf[b], [pat0 + off], v0 + p0)
                plsc.store_scatter(obuf[b], [pat1 + off], v1 + p1)

        def out_copies(j, b):
            l, q = (u0 + j) // nq, (u0 + j) % nq
            return [
                pltpu.make_async_copy(
                    obuf[b].at[pl.ds(vb * osz, osz)],
                    out_h.at[l, vb, pl.ds(q * osz, osz)],
                    osem[b][vb])
                for vb in range(VB)
            ]

        def start_out(j, b):
            for c in out_copies(j, b):
                c.start()

        def wait_out(j, b):
            for c in out_copies(j, b):
                c.wait()

        stage(0, 0)
        stage(1, 1)
        for j in range(NB):
            b = j % NB
            wait_gather(b)
            transpose_add(j, b)
            stage(j + NB, b)
            start_out(j, b)

        def group(g, c):
            for b in range(NB):
                j = g * NB + b
                wait_gather(b)
                wait_out(j - NB, b)
                transpose_add(j, b)
                stage(j + NB, b)
                start_out(j, b)
            return c

        lax.fori_loop(1, upw // NB - 1, group, 0)

        for j in range(upw - NB, upw):
            b = j % NB
            wait_gather(b)
            wait_out(j - NB, b)
            transpose_add(j, b)
            start_out(j, b)
        for j in range(upw - NB, upw):
            wait_out(j, j % NB)

    return k(xt, table, pos)


def kernel(x, table, pos_emb):
    xt = jnp.transpose(x).astype(jnp.int32)
    o5 = _sc_embed(xt, table, pos_emb[:L])
    o = o5.reshape(L, VB, B // 128, 8, 128).transpose(2, 4, 0, 1, 3)
    return o.reshape(B, L, D)


# speedup vs baseline: 1.4430x; 1.0303x over previous
import functools

import jax
import jax.numpy as jnp
from jax import lax
from jax.experimental import pallas as pl
from jax.experimental.pallas import tpu as pltpu
from jax.experimental.pallas import tpu_sc as plsc

NUM_CORES = 2
NUM_SUBCORES = 16
LANES = 16
QB = 256
NB = 4

B = 4096
L = 200
D = 32
VB = D // 8
LG = 25
BG = 1024


@jax.jit
def _sc_embed(xt, table, pos):
    d2 = (B // 128) * 8 * 128
    osz = (QB // 128) * 8 * 128
    nq = BG // QB
    upw = LG * nq

    mesh = plsc.VectorSubcoreMesh(
        core_axis_name="c", subcore_axis_name="s",
        num_cores=NUM_CORES, num_subcores=NUM_SUBCORES,
    )

    @functools.partial(
        pl.kernel,
        mesh=mesh,
        out_type=jax.ShapeDtypeStruct((L, VB, d2), jnp.float32),
        scratch_types=[
            pltpu.VMEM((LG, BG), jnp.int32),
            pltpu.VMEM((LG, D), jnp.float32),
            [pltpu.VMEM((QB, D), jnp.float32) for _ in range(NB)],
            [pltpu.VMEM((VB * osz,), jnp.float32) for _ in range(NB)],
            [pltpu.SemaphoreType.DMA for _ in range(NB)],
            [[pltpu.SemaphoreType.DMA for _ in range(VB)] for _ in range(NB)],
        ],
        compiler_params=pltpu.CompilerParams(
            use_tc_tiling_on_sc=False, needs_layout_passes=False),
    )
    def k(xt_h, t_h, pos_h, out_h, idx_all, pos_v, rows, obuf, gsem, osem):
        wid = lax.axis_index("s") * NUM_CORES + lax.axis_index("c")
        gi = lax.shift_right_logical(wid, 2)
        gj = lax.bitwise_and(wid, 3)
        l0 = gi * LG
        iota = lax.iota(jnp.int32, LANES)
        pat0 = lax.shift_right_logical(iota, 3) * osz + \
            lax.bitwise_and(iota, 7) * 128
        pat1 = pat0 + 2 * osz

        pltpu.sync_copy(xt_h.at[pl.ds(l0, LG), pl.ds(gj * BG, BG)], idx_all)
        pltpu.sync_copy(pos_h.at[pl.ds(l0, LG)], pos_v)

        def unit_idx(j):
            return lax.shift_right_logical(j, 2), lax.bitwise_and(j, 3)

        def gather_op(j, b):
            lloc, qloc = unit_idx(j)
            return pltpu.make_async_copy(
                t_h.at[idx_all.at[lloc, pl.ds(qloc * QB, QB)]],
                rows[b], gsem[b])

        def stage(j, b):
            gather_op(j, b).start()

        def wait_gather(j, b):
            gather_op(j, b).wait()

        def transpose_add(j, b):
            lloc, _ = unit_idx(j)
            p0 = pos_v[lloc, pl.ds(0, LANES)]
            p1 = pos_v[lloc, pl.ds(LANES, LANES)]

            @plsc.parallel_loop(0, QB, unroll=8)
            def jbody(jj):
                off = lax.shift_right_logical(jj, 7) * 1024 + \
                    lax.bitwise_and(jj, 127)
                v0 = rows[b][jj, pl.ds(0, LANES)]
                v1 = rows[b][jj, pl.ds(LANES, LANES)]
                plsc.store_scatter(obuf[b], [pat0 + off], v0 + p0)
                plsc.store_scatter(obuf[b], [pat1 + off], v1 + p1)

        def out_copies(j, b):
            lloc, qloc = unit_idx(j)
            qg = gj * nq + qloc
            return [
                pltpu.make_async_copy(
                    obuf[b].at[pl.ds(vb * osz, osz)],
                    out_h.at[l0 + lloc, vb, pl.ds(qg * osz, osz)],
                    osem[b][vb])
                for vb in range(VB)
            ]

        def start_out(j, b):
            for c in out_copies(j, b):
                c.start()

        def wait_out(j, b):
            for c in out_copies(j, b):
                c.wait()

        for b in range(NB):
            stage(b, b)
        for j in range(NB):
            b = j % NB
            wait_gather(j, b)
            transpose_add(j, b)
            stage(j + NB, b)
            start_out(j, b)

        def group(g, c):
            for b in range(NB):
                j = g * NB + b
                wait_gather(j, b)
                wait_out(j - NB, b)
                transpose_add(j, b)
                stage(j + NB, b)
                start_out(j, b)
            return c

        lax.fori_loop(1, upw // NB - 1, group, 0)

        for j in range(upw - NB, upw):
            b = j % NB
            wait_gather(j, b)
            wait_out(j - NB, b)
            transpose_add(j, b)
            start_out(j, b)
        for j in range(upw - NB, upw):
            wait_out(j, j % NB)

    return k(xt, table, pos)


def kernel(x, table, pos_emb):
    xt = jnp.transpose(x).astype(jnp.int32)
    o5 = _sc_embed(xt, table, pos_emb[:L])
    o = o5.reshape(L, VB, B // 128, 8, 128).transpose(2, 4, 0, 1, 3)
    return o.reshape(B, L, D)


# speedup vs baseline: 1.4449x; 1.0013x over previous
import functools

import jax
import jax.numpy as jnp
from jax import lax
from jax.experimental import pallas as pl
from jax.experimental.pallas import tpu as pltpu
from jax.experimental.pallas import tpu_sc as plsc

NUM_CORES = 2
NUM_SUBCORES = 16
LANES = 16
QB = 256
NB = 5

B = 4096
L = 200
D = 32
VB = D // 8
LG = 25
BG = 1024


@jax.jit
def _sc_embed(xt, table, pos):
    d2 = (B // 128) * 8 * 128
    osz = (QB // 128) * 8 * 128
    nq = BG // QB
    upw = LG * nq

    mesh = plsc.VectorSubcoreMesh(
        core_axis_name="c", subcore_axis_name="s",
        num_cores=NUM_CORES, num_subcores=NUM_SUBCORES,
    )

    @functools.partial(
        pl.kernel,
        mesh=mesh,
        out_type=jax.ShapeDtypeStruct((L, VB, d2), jnp.float32),
        scratch_types=[
            pltpu.VMEM((LG, BG), jnp.int32),
            pltpu.VMEM((LG, D), jnp.float32),
            [pltpu.VMEM((QB, D), jnp.float32) for _ in range(NB)],
            [pltpu.VMEM((VB * osz,), jnp.float32) for _ in range(NB)],
            [pltpu.SemaphoreType.DMA for _ in range(NB)],
            [[pltpu.SemaphoreType.DMA for _ in range(VB)] for _ in range(NB)],
        ],
        compiler_params=pltpu.CompilerParams(
            use_tc_tiling_on_sc=False, needs_layout_passes=False),
    )
    def k(xt_h, t_h, pos_h, out_h, idx_all, pos_v, rows, obuf, gsem, osem):
        wid = lax.axis_index("s") * NUM_CORES + lax.axis_index("c")
        gi = lax.shift_right_logical(wid, 2)
        gj = lax.bitwise_and(wid, 3)
        l0 = gi * LG
        iota = lax.iota(jnp.int32, LANES)
        pat0 = lax.shift_right_logical(iota, 3) * osz + \
            lax.bitwise_and(iota, 7) * 128
        pat1 = pat0 + 2 * osz

        pltpu.sync_copy(xt_h.at[pl.ds(l0, LG), pl.ds(gj * BG, BG)], idx_all)
        pltpu.sync_copy(pos_h.at[pl.ds(l0, LG)], pos_v)

        def unit_idx(j):
            return lax.shift_right_logical(j, 2), lax.bitwise_and(j, 3)

        def gather_op(j, b):
            lloc, qloc = unit_idx(j)
            return pltpu.make_async_copy(
                t_h.at[idx_all.at[lloc, pl.ds(qloc * QB, QB)]],
                rows[b], gsem[b])

        def stage(j, b):
            gather_op(j, b).start()

        def wait_gather(j, b):
            gather_op(j, b).wait()

        def transpose_add(j, b):
            lloc, _ = unit_idx(j)
            p0 = pos_v[lloc, pl.ds(0, LANES)]
            p1 = pos_v[lloc, pl.ds(LANES, LANES)]

            @plsc.parallel_loop(0, QB, unroll=8)
            def jbody(jj):
                off = lax.shift_right_logical(jj, 7) * 1024 + \
                    lax.bitwise_and(jj, 127)
                v0 = rows[b][jj, pl.ds(0, LANES)]
                v1 = rows[b][jj, pl.ds(LANES, LANES)]
                plsc.store_scatter(obuf[b], [pat0 + off], v0 + p0)
                plsc.store_scatter(obuf[b], [pat1 + off], v1 + p1)

        def out_copies(j, b):
            lloc, qloc = unit_idx(j)
            qg = gj * nq + qloc
            return [
                pltpu.make_async_copy(
                    obuf[b].at[pl.ds(vb * osz, osz)],
                    out_h.at[l0 + lloc, vb, pl.ds(qg * osz, osz)],
                    osem[b][vb])
                for vb in range(VB)
            ]

        def start_out(j, b):
            for c in out_copies(j, b):
                c.start()

        def wait_out(j, b):
            for c in out_copies(j, b):
                c.wait()

        for b in range(NB):
            stage(b, b)
        for j in range(NB):
            b = j % NB
            wait_gather(j, b)
            transpose_add(j, b)
            stage(j + NB, b)
            start_out(j, b)

        def group(g, c):
            for b in range(NB):
                j = g * NB + b
                wait_gather(j, b)
                wait_out(j - NB, b)
                transpose_add(j, b)
                stage(j + NB, b)
                start_out(j, b)
            return c

        lax.fori_loop(1, upw // NB - 1, group, 0)

        for j in range(upw - NB, upw):
            b = j % NB
            wait_gather(j, b)
            wait_out(j - NB, b)
            transpose_add(j, b)
            start_out(j, b)
        for j in range(upw - NB, upw):
            wait_out(j, j % NB)

    return k(xt, table, pos)


def kernel(x, table, pos_emb):
    xt = jnp.transpose(x).astype(jnp.int32)
    o5 = _sc_embed(xt, table, pos_emb[:L])
    o = o5.reshape(L, VB, B // 128, 8, 128).transpose(2, 4, 0, 1, 3)
    return o.reshape(B, L, D)
